# baseline (speedup 1.0000x reference)
import jax
import jax.numpy as jnp
from jax.experimental import pallas as pl

_SCALE = 0.707106781
_BH = 8
_BW = 256


def _body(h_ref, w_ref, o_ref):
    hs = h_ref[...] * _SCALE
    ws = w_ref[...] * _SCALE
    o_ref[...] = hs[:, None, :] + ws[None, :, :]


def kernel(height, width, h_embed, w_embed):
    max_h, dim = h_embed.shape
    max_w = w_embed.shape[0]
    return pl.pallas_call(
        _body,
        grid=(max_h // _BH, max_w // _BW),
        in_specs=[
            pl.BlockSpec((_BH, dim), lambda i, j: (i, 0)),
            pl.BlockSpec((_BW, dim), lambda i, j: (j, 0)),
        ],
        out_specs=pl.BlockSpec((_BH, _BW, dim), lambda i, j: (i, j, 0)),
        out_shape=jax.ShapeDtypeStruct((max_h, max_w, dim), jnp.float32),
    )(h_embed, w_embed)


# speedup vs baseline: 1.6599x; 1.6599x over previous
import jax
import jax.numpy as jnp
from jax.experimental import pallas as pl

_SCALE = 0.707106781


def _body(h_ref, w_ref, o_ref):
    hs = h_ref[...] * _SCALE
    ws = w_ref[...] * _SCALE
    o_ref[...] = hs[:, None, :] + ws[None, :, :]


def kernel(height, width, h_embed, w_embed):
    max_h, dim = h_embed.shape
    max_w = w_embed.shape[0]
    bh = 8
    return pl.pallas_call(
        _body,
        grid=(max_h // bh,),
        in_specs=[
            pl.BlockSpec((bh, dim), lambda i: (i, 0)),
            pl.BlockSpec((max_w, dim), lambda i: (0, 0)),
        ],
        out_specs=pl.BlockSpec((bh, max_w, dim), lambda i: (i, 0, 0)),
        out_shape=jax.ShapeDtypeStruct((max_h, max_w, dim), jnp.float32),
    )(h_embed, w_embed)
